# baseline (speedup 1.0000x reference)
import jax
import jax.numpy as jnp
from jax import lax
from jax.experimental import pallas as pl
from jax.experimental.pallas import tpu as pltpu
from jax.experimental.pallas import tpu_sc as plsc

_BATCH = 4096
_HIST = 50
_D = 64
_B = _BATCH * _HIST
_NC = 2
_NS = 16
_NW = _NC * _NS
_HB = _BATCH // 2
_PB = 128
_JOBS_PER_T = _HB // _PB
_NJOBS = _HIST * 2 * _JOBS_PER_T
_JPW = _NJOBS // _NW
_GJ = 2
_NG = _JPW // _GJ


def _emb_body(idx_hbm, table_hbm, out_hbm, idx_v, rows_a, rows_b,
              isem, gsem_a, gsem_b, wsem_a, wsem_b):
  wid = lax.axis_index("s") * _NC + lax.axis_index("c")
  rows = (rows_a, rows_b)
  gsem = (gsem_a, gsem_b)
  wsem = (wsem_a, wsem_b)

  base = wid * _JPW * _PB
  pltpu.sync_copy(idx_hbm.at[pl.ds(base, _JPW * _PB)], idx_v)

  def start_gather(g):
    b = g % 2
    return pltpu.async_copy(
        table_hbm.at[idx_v.at[pl.ds(g * _GJ * _PB, _GJ * _PB)]],
        rows[b], gsem[b])

  def start_writes(g):
    b = g % 2
    ws = []
    for u in range(_GJ):
      j = wid * _JPW + g * _GJ + u
      t = j // (2 * _JOBS_PER_T)
      r = j % (2 * _JOBS_PER_T)
      h, pb = r // _JOBS_PER_T, r % _JOBS_PER_T
      ws.append(pltpu.async_copy(
          rows[b].at[pl.ds(u * _PB, _PB), pl.ds(0, _D)],
          out_hbm.at[t, pl.ds(pb * _PB, _PB), pl.ds(h * _D, _D)], wsem[b]))
    return ws

  gathers = [None] * _NG
  writes = [None] * _NG
  gathers[0] = start_gather(0)
  for g in range(_NG):
    gathers[g].wait()
    if g + 1 < _NG:
      if g >= 1:
        for w in writes[g - 1]:
          w.wait()
      gathers[g + 1] = start_gather(g + 1)
    writes[g] = start_writes(g)
  for w in writes[_NG - 2] + writes[_NG - 1]:
    w.wait()


_emb_call = pl.kernel(
    _emb_body,
    out_type=jax.ShapeDtypeStruct((_HIST, _HB, 2 * _D), jnp.float32),
    mesh=plsc.VectorSubcoreMesh(core_axis_name="c", subcore_axis_name="s"),
    scratch_types=[
        pltpu.VMEM((_JPW * _PB,), jnp.int32),
        pltpu.VMEM((_GJ * _PB, 2 * _D), jnp.float32),
        pltpu.VMEM((_GJ * _PB, 2 * _D), jnp.float32),
        pltpu.SemaphoreType.DMA,
        pltpu.SemaphoreType.DMA,
        pltpu.SemaphoreType.DMA,
        pltpu.SemaphoreType.DMA,
        pltpu.SemaphoreType.DMA,
    ],
    compiler_params=pltpu.CompilerParams(use_tc_tiling_on_sc=False),
)


@jax.jit
def kernel(input, weights):
  idx_t = input.astype(jnp.int32).T.reshape(_B)
  wpad = jnp.pad(weights, ((0, 0), (0, _D)))
  packed = _emb_call(idx_t, wpad)
  x = packed.reshape(_HIST, _HB, 2, _D)
  return x.transpose(2, 1, 0, 3).reshape(_BATCH, _HIST, _D)


# speedup vs baseline: 1.1104x; 1.1104x over previous
import jax
import jax.numpy as jnp
from jax import lax
from jax.experimental import pallas as pl
from jax.experimental.pallas import tpu as pltpu
from jax.experimental.pallas import tpu_sc as plsc

_BATCH = 4096
_HIST = 50
_D = 64
_B = _BATCH * _HIST
_NC = 2
_NS = 16
_NW = _NC * _NS
_HB = _BATCH // 2
_PB = 128
_JOBS_PER_T = _HB // _PB
_NJOBS = _HIST * 2 * _JOBS_PER_T
_JPW = _NJOBS // _NW
_GJ = 5
_NG = _JPW // _GJ


def _emb_body(idx_hbm, table_hbm, out_hbm, idx_v, rows_a, rows_b,
              isem, gsem_a, gsem_b, wsem_a, wsem_b):
  wid = lax.axis_index("s") * _NC + lax.axis_index("c")
  rows = (rows_a, rows_b)
  gsem = (gsem_a, gsem_b)
  wsem = (wsem_a, wsem_b)

  base = wid * _JPW * _PB
  pltpu.sync_copy(idx_hbm.at[pl.ds(base, _JPW * _PB)], idx_v)

  def start_gather(g):
    b = g % 2
    return pltpu.async_copy(
        table_hbm.at[idx_v.at[pl.ds(g * _GJ * _PB, _GJ * _PB)]],
        rows[b], gsem[b])

  def start_writes(g):
    b = g % 2
    ws = []
    for u in range(_GJ):
      j = wid * _JPW + g * _GJ + u
      t = j // (2 * _JOBS_PER_T)
      r = j % (2 * _JOBS_PER_T)
      h, pb = r // _JOBS_PER_T, r % _JOBS_PER_T
      ws.append(pltpu.async_copy(
          rows[b].at[pl.ds(u * _PB, _PB), :],
          out_hbm.at[t, pl.ds(pb * _PB, _PB), pl.ds(h * _D, _D)], wsem[b]))
    return ws

  gathers = [None] * _NG
  writes = [None] * _NG
  gathers[0] = start_gather(0)
  for g in range(_NG):
    if g + 1 < _NG:
      if g >= 1:
        for w in writes[g - 1]:
          w.wait()
      gathers[g + 1] = start_gather(g + 1)
    gathers[g].wait()
    writes[g] = start_writes(g)
  for w in writes[_NG - 2] + writes[_NG - 1]:
    w.wait()


_emb_call = pl.kernel(
    _emb_body,
    out_type=jax.ShapeDtypeStruct((_HIST, _HB, 2 * _D), jnp.float32),
    mesh=plsc.VectorSubcoreMesh(core_axis_name="c", subcore_axis_name="s"),
    scratch_types=[
        pltpu.VMEM((_JPW * _PB,), jnp.int32),
        pltpu.VMEM((_GJ * _PB, _D), jnp.float32),
        pltpu.VMEM((_GJ * _PB, _D), jnp.float32),
        pltpu.SemaphoreType.DMA,
        pltpu.SemaphoreType.DMA,
        pltpu.SemaphoreType.DMA,
        pltpu.SemaphoreType.DMA,
        pltpu.SemaphoreType.DMA,
    ],
    compiler_params=pltpu.CompilerParams(use_tc_tiling_on_sc=False),
)


@jax.jit
def kernel(input, weights):
  idx_t = input.astype(jnp.int32).T.reshape(_B)
  packed = _emb_call(idx_t, weights)
  x = packed.reshape(_HIST, _HB, 2, _D)
  return x.transpose(2, 1, 0, 3).reshape(_BATCH, _HIST, _D)
